# baseline (speedup 1.0000x reference)
import functools

import jax
import jax.numpy as jnp
from jax.experimental import pallas as pl
from jax.experimental.pallas import tpu as pltpu

_TEMPERATURE = 0.8
_TOP_K = 50
_TOP_P = 0.9

_CHUNK = 125
_BLOCK_ROWS = 16
_SLOTS = 128
_IMAX = 2**31 - 1
_BW = _TOP_K * _CHUNK


def _cumsum_lanes(a):
    sh = 1
    while sh < _SLOTS:
        pad = jnp.zeros(a.shape[:-1] + (sh,), a.dtype)
        a = a + jnp.concatenate([pad, a[:, :-sh]], axis=1)
        sh *= 2
    return a


def _threefry_gumbel(lin):
    ks0 = jnp.uint32(0)
    ks1 = jnp.uint32(42)
    ks2 = ks0 ^ ks1 ^ jnp.uint32(0x1BD11BDA)
    ks = (ks0, ks1, ks2)
    rotations = ((13, 15, 26, 6), (17, 29, 16, 24))
    x0 = jnp.zeros_like(lin) + ks0
    x1 = lin + ks1
    for i in range(5):
        for r in rotations[i % 2]:
            x0 = x0 + x1
            x1 = (x1 << jnp.uint32(r)) | (x1 >> jnp.uint32(32 - r))
            x1 = x1 ^ x0
        x0 = x0 + ks[(i + 1) % 3]
        x1 = x1 + ks[(i + 2) % 3] + jnp.uint32(i + 1)
    bits = x0 ^ x1
    fl = jax.lax.bitcast_convert_type(
        (bits >> jnp.uint32(9)) | jnp.uint32(0x3F800000), jnp.float32
    ) - jnp.float32(1.0)
    u = jnp.maximum(
        jnp.float32(1e-9),
        fl * jnp.float32(1.0 - 1e-9) + jnp.float32(1e-9),
    )
    return -jnp.log(-jnp.log(u))


def _sampler_kernel(x_ref, probs_ref, tok_ref, buf_ref, buf3_ref):
    r = x_ref.shape[0]
    nchunk = x_ref.shape[1]
    vocab = nchunk * x_ref.shape[2]
    i = pl.program_id(0)

    cm = jnp.max(x_ref[...], axis=2)
    chunk_iota = jax.lax.broadcasted_iota(jnp.int32, cm.shape, 1)
    slot_iota = jax.lax.broadcasted_iota(jnp.int32, (r, _SLOTS), 1)

    def rank_body(j, carry):
        cm_c, idxs = carry
        m = jnp.max(cm_c, axis=1, keepdims=True)
        ii = jnp.min(
            jnp.where(cm_c == m, chunk_iota, _IMAX), axis=1, keepdims=True
        )
        cm_c = jnp.where(chunk_iota == ii, -jnp.inf, cm_c)
        idxs = jnp.where(slot_iota == j, ii, idxs)
        return cm_c, idxs

    _, idxs = jax.lax.fori_loop(
        0, _TOP_K, rank_body, (cm, jnp.zeros((r, _SLOTS), jnp.int32))
    )

    onehot = (
        idxs[:, :_TOP_K, None]
        == jax.lax.broadcasted_iota(jnp.int32, (r, _TOP_K, nchunk), 2)
    ).astype(jnp.float32)
    for rr in range(_BLOCK_ROWS):
        buf3_ref[rr] = jax.lax.dot_general(
            onehot[rr],
            x_ref[rr],
            (((1,), (0,)), ((), ())),
            precision=jax.lax.Precision.HIGHEST,
            preferred_element_type=jnp.float32,
        )
    buf_ref[...] = jnp.reshape(buf3_ref[...], (r, _BW)) / _TEMPERATURE

    scaled_buf = buf_ref[...]

    def body(j, carry):
        vals, cnts = carry
        y = buf_ref[...]
        m = jnp.max(y, axis=1, keepdims=True)
        eq = y == m
        c = jnp.sum(jnp.where(eq, 1.0, 0.0), axis=1, keepdims=True)
        buf_ref[...] = jnp.where(eq, -jnp.inf, y)
        sel = slot_iota == j
        vals = jnp.where(sel, m, vals)
        cnts = jnp.where(sel, c, cnts)
        return vals, cnts

    vals0 = jnp.full((r, _SLOTS), -jnp.inf, jnp.float32)
    cnts0 = jnp.zeros((r, _SLOTS), jnp.float32)
    vals, cnts = jax.lax.fori_loop(0, _TOP_K, body, (vals0, cnts0))

    m_row = vals[:, 0:1]
    wgt = cnts * jnp.exp(vals - m_row)
    cumc = _cumsum_lanes(cnts)
    cumw = _cumsum_lanes(wgt)
    cumc_excl = cumc - cnts
    s_excl = cumw - wgt
    topk_keep = cumc_excl < float(_TOP_K)
    z_topk = jnp.sum(jnp.where(topk_keep, wgt, 0.0), axis=1, keepdims=True)
    keep = jnp.logical_and(topk_keep, s_excl <= _TOP_P * z_topk)
    thresh = jnp.min(jnp.where(keep, vals, jnp.inf), axis=1, keepdims=True)
    z_final = jnp.sum(jnp.where(keep, wgt, 0.0), axis=1, keepdims=True)

    lane_iota = jax.lax.broadcasted_iota(jnp.int32, (r, _BW), 1)
    slot_of = lane_iota // _CHUNK
    lane_in = lane_iota - slot_of * _CHUNK
    chunk_id = jnp.zeros((r, _BW), jnp.int32)
    for s in range(_TOP_K):
        chunk_id = jnp.where(slot_of == s, idxs[:, s : s + 1], chunk_id)
    vidx = chunk_id * _CHUNK + lane_in
    row_glob = i * _BLOCK_ROWS + jax.lax.broadcasted_iota(
        jnp.int32, (r, _BW), 0
    )
    lin = (row_glob * vocab + vidx).astype(jnp.uint32)
    g = _threefry_gumbel(lin)
    log_z = jnp.log(z_final)
    log_eps = jnp.log(jnp.float32(1e-12))
    kept_buf = scaled_buf >= thresh
    score = jnp.where(kept_buf, scaled_buf - m_row - log_z, log_eps) + g
    smax = jnp.max(score, axis=1, keepdims=True)
    tok_ref[...] = jnp.min(
        jnp.where(score == smax, vidx, _IMAX), axis=1, keepdims=True
    )

    scaled = x_ref[...] / _TEMPERATURE
    e = jnp.exp(scaled - m_row[:, :, None])
    kept = scaled >= thresh[:, :, None]
    inv_z = (1.0 / z_final)[:, :, None]
    probs_ref[...] = jnp.where(kept, e * inv_z, 0.0)


@functools.partial(jax.jit)
def kernel(logits):
    rows, vocab = logits.shape
    nchunk = vocab // _CHUNK
    x3 = jnp.reshape(logits, (rows, nchunk, _CHUNK))
    grid = (rows // _BLOCK_ROWS,)

    probs3, tok = pl.pallas_call(
        _sampler_kernel,
        grid=grid,
        in_specs=[
            pl.BlockSpec((_BLOCK_ROWS, nchunk, _CHUNK), lambda i: (i, 0, 0)),
        ],
        out_specs=[
            pl.BlockSpec((_BLOCK_ROWS, nchunk, _CHUNK), lambda i: (i, 0, 0)),
            pl.BlockSpec((_BLOCK_ROWS, 1), lambda i: (i, 0)),
        ],
        out_shape=[
            jax.ShapeDtypeStruct((rows, nchunk, _CHUNK), jnp.float32),
            jax.ShapeDtypeStruct((rows, 1), jnp.int32),
        ],
        scratch_shapes=[
            pltpu.VMEM((_BLOCK_ROWS, _BW), jnp.float32),
            pltpu.VMEM((_BLOCK_ROWS, _TOP_K, _CHUNK), jnp.float32),
        ],
    )(x3)
    return jnp.reshape(probs3, (rows, vocab)), jnp.reshape(tok, (rows,))


# speedup vs baseline: 1.0100x; 1.0100x over previous
import functools

import jax
import jax.numpy as jnp
from jax.experimental import pallas as pl
from jax.experimental.pallas import tpu as pltpu

_TEMPERATURE = 0.8
_TOP_K = 50
_TOP_P = 0.9

_CHUNK = 100
_BLOCK_ROWS = 16
_SLOTS = 128
_IMAX = 2**31 - 1
_BW = _TOP_K * _CHUNK


def _cumsum_lanes(a):
    sh = 1
    while sh < _SLOTS:
        pad = jnp.zeros(a.shape[:-1] + (sh,), a.dtype)
        a = a + jnp.concatenate([pad, a[:, :-sh]], axis=1)
        sh *= 2
    return a


def _threefry_gumbel(lin):
    ks0 = jnp.uint32(0)
    ks1 = jnp.uint32(42)
    ks2 = ks0 ^ ks1 ^ jnp.uint32(0x1BD11BDA)
    ks = (ks0, ks1, ks2)
    rotations = ((13, 15, 26, 6), (17, 29, 16, 24))
    x0 = jnp.zeros_like(lin) + ks0
    x1 = lin + ks1
    for i in range(5):
        for r in rotations[i % 2]:
            x0 = x0 + x1
            x1 = (x1 << jnp.uint32(r)) | (x1 >> jnp.uint32(32 - r))
            x1 = x1 ^ x0
        x0 = x0 + ks[(i + 1) % 3]
        x1 = x1 + ks[(i + 2) % 3] + jnp.uint32(i + 1)
    bits = x0 ^ x1
    fl = jax.lax.bitcast_convert_type(
        (bits >> jnp.uint32(9)) | jnp.uint32(0x3F800000), jnp.float32
    ) - jnp.float32(1.0)
    u = jnp.maximum(
        jnp.float32(1e-9),
        fl * jnp.float32(1.0 - 1e-9) + jnp.float32(1e-9),
    )
    return -jnp.log(-jnp.log(u))


def _chunk_rank_kernel(x_ref, cidx_ref):
    r = x_ref.shape[0]
    cm = jnp.max(x_ref[...], axis=2)
    chunk_iota = jax.lax.broadcasted_iota(jnp.int32, cm.shape, 1)
    slot_iota = jax.lax.broadcasted_iota(jnp.int32, (r, _SLOTS), 1)

    def body(j, carry):
        cm_c, idxs = carry
        m = jnp.max(cm_c, axis=1, keepdims=True)
        ii = jnp.min(
            jnp.where(cm_c == m, chunk_iota, _IMAX), axis=1, keepdims=True
        )
        cm_c = jnp.where(chunk_iota == ii, -jnp.inf, cm_c)
        idxs = jnp.where(slot_iota == j, ii, idxs)
        return cm_c, idxs

    _, idxs = jax.lax.fori_loop(
        0, _TOP_K, body, (cm, jnp.zeros((r, _SLOTS), jnp.int32))
    )
    cidx_ref[...] = idxs


def _sampler_kernel(cidx_sref, x_ref, cvec_ref, probs_ref, tok_ref, buf_ref):
    r = x_ref.shape[0]
    vocab = x_ref.shape[1] * x_ref.shape[2]
    i = pl.program_id(0)

    for s in range(_TOP_K):
        for rr in range(_BLOCK_ROWS):
            c = cidx_sref[i * _BLOCK_ROWS + rr, s]
            buf_ref[pl.ds(rr, 1), pl.ds(s * _CHUNK, _CHUNK)] = (
                x_ref[rr, pl.ds(c, 1), :] / _TEMPERATURE
            )

    scaled_buf = buf_ref[...]
    slot_iota = jax.lax.broadcasted_iota(jnp.int32, (r, _SLOTS), 1)

    def body(j, carry):
        vals, cnts = carry
        y = buf_ref[...]
        m = jnp.max(y, axis=1, keepdims=True)
        eq = y == m
        c = jnp.sum(jnp.where(eq, 1.0, 0.0), axis=1, keepdims=True)
        buf_ref[...] = jnp.where(eq, -jnp.inf, y)
        sel = slot_iota == j
        vals = jnp.where(sel, m, vals)
        cnts = jnp.where(sel, c, cnts)
        return vals, cnts

    vals0 = jnp.full((r, _SLOTS), -jnp.inf, jnp.float32)
    cnts0 = jnp.zeros((r, _SLOTS), jnp.float32)
    vals, cnts = jax.lax.fori_loop(0, _TOP_K, body, (vals0, cnts0))

    m_row = vals[:, 0:1]
    wgt = cnts * jnp.exp(vals - m_row)
    cumc = _cumsum_lanes(cnts)
    cumw = _cumsum_lanes(wgt)
    cumc_excl = cumc - cnts
    s_excl = cumw - wgt
    topk_keep = cumc_excl < float(_TOP_K)
    z_topk = jnp.sum(jnp.where(topk_keep, wgt, 0.0), axis=1, keepdims=True)
    keep = jnp.logical_and(topk_keep, s_excl <= _TOP_P * z_topk)
    thresh = jnp.min(jnp.where(keep, vals, jnp.inf), axis=1, keepdims=True)
    z_final = jnp.sum(jnp.where(keep, wgt, 0.0), axis=1, keepdims=True)

    lane_iota = jax.lax.broadcasted_iota(jnp.int32, (r, _BW), 1)
    slot_of = lane_iota // _CHUNK
    lane_in = lane_iota - slot_of * _CHUNK
    cvec = cvec_ref[...]
    chunk_id = jnp.zeros((r, _BW), jnp.int32)
    for s in range(_TOP_K):
        chunk_id = jnp.where(slot_of == s, cvec[:, s : s + 1], chunk_id)
    vidx = chunk_id * _CHUNK + lane_in
    row_glob = i * _BLOCK_ROWS + jax.lax.broadcasted_iota(
        jnp.int32, (r, _BW), 0
    )
    lin = (row_glob * vocab + vidx).astype(jnp.uint32)
    g = _threefry_gumbel(lin)
    log_z = jnp.log(z_final)
    log_eps = jnp.log(jnp.float32(1e-12))
    kept_buf = scaled_buf >= thresh
    score = jnp.where(kept_buf, scaled_buf - m_row - log_z, log_eps) + g
    smax = jnp.max(score, axis=1, keepdims=True)
    tok_ref[...] = jnp.min(
        jnp.where(score == smax, vidx, _IMAX), axis=1, keepdims=True
    )

    scaled = x_ref[...] / _TEMPERATURE
    e = jnp.exp(scaled - m_row[:, :, None])
    kept = scaled >= thresh[:, :, None]
    inv_z = (1.0 / z_final)[:, :, None]
    probs_ref[...] = jnp.where(kept, e * inv_z, 0.0)


@functools.partial(jax.jit)
def kernel(logits):
    rows, vocab = logits.shape
    nchunk = vocab // _CHUNK
    x3 = jnp.reshape(logits, (rows, nchunk, _CHUNK))
    grid = (rows // _BLOCK_ROWS,)

    cidx = pl.pallas_call(
        _chunk_rank_kernel,
        grid=grid,
        in_specs=[pl.BlockSpec((_BLOCK_ROWS, nchunk, _CHUNK), lambda i: (i, 0, 0))],
        out_specs=pl.BlockSpec((_BLOCK_ROWS, _SLOTS), lambda i: (i, 0)),
        out_shape=jax.ShapeDtypeStruct((rows, _SLOTS), jnp.int32),
    )(x3)

    grid_spec = pltpu.PrefetchScalarGridSpec(
        num_scalar_prefetch=1,
        grid=grid,
        in_specs=[
            pl.BlockSpec((_BLOCK_ROWS, nchunk, _CHUNK), lambda i, s: (i, 0, 0)),
            pl.BlockSpec((_BLOCK_ROWS, _SLOTS), lambda i, s: (i, 0)),
        ],
        out_specs=[
            pl.BlockSpec((_BLOCK_ROWS, nchunk, _CHUNK), lambda i, s: (i, 0, 0)),
            pl.BlockSpec((_BLOCK_ROWS, 1), lambda i, s: (i, 0)),
        ],
        scratch_shapes=[pltpu.VMEM((_BLOCK_ROWS, _BW), jnp.float32)],
    )
    probs3, tok = pl.pallas_call(
        _sampler_kernel,
        grid_spec=grid_spec,
        out_shape=[
            jax.ShapeDtypeStruct((rows, nchunk, _CHUNK), jnp.float32),
            jax.ShapeDtypeStruct((rows, 1), jnp.int32),
        ],
    )(cidx, x3, cidx)
    return jnp.reshape(probs3, (rows, vocab)), jnp.reshape(tok, (rows,))


# speedup vs baseline: 1.0352x; 1.0249x over previous
import functools

import jax
import jax.numpy as jnp
from jax.experimental import pallas as pl
from jax.experimental.pallas import tpu as pltpu

_TEMPERATURE = 0.8
_TOP_K = 50
_TOP_P = 0.9

_CHUNK = 125
_BLOCK_ROWS = 16
_SLOTS = 128
_IMAX = 2**31 - 1
_BW = _TOP_K * _CHUNK


def _cumsum_lanes(a):
    sh = 1
    while sh < _SLOTS:
        pad = jnp.zeros(a.shape[:-1] + (sh,), a.dtype)
        a = a + jnp.concatenate([pad, a[:, :-sh]], axis=1)
        sh *= 2
    return a


def _threefry_gumbel(lin):
    ks0 = jnp.uint32(0)
    ks1 = jnp.uint32(42)
    ks2 = ks0 ^ ks1 ^ jnp.uint32(0x1BD11BDA)
    ks = (ks0, ks1, ks2)
    rotations = ((13, 15, 26, 6), (17, 29, 16, 24))
    x0 = jnp.zeros_like(lin) + ks0
    x1 = lin + ks1
    for i in range(5):
        for r in rotations[i % 2]:
            x0 = x0 + x1
            x1 = (x1 << jnp.uint32(r)) | (x1 >> jnp.uint32(32 - r))
            x1 = x1 ^ x0
        x0 = x0 + ks[(i + 1) % 3]
        x1 = x1 + ks[(i + 2) % 3] + jnp.uint32(i + 1)
    bits = x0 ^ x1
    fl = jax.lax.bitcast_convert_type(
        (bits >> jnp.uint32(9)) | jnp.uint32(0x3F800000), jnp.float32
    ) - jnp.float32(1.0)
    u = jnp.maximum(
        jnp.float32(1e-9),
        fl * jnp.float32(1.0 - 1e-9) + jnp.float32(1e-9),
    )
    return -jnp.log(-jnp.log(u))


def _chunk_rank_kernel(x_ref, cidx_ref):
    r = x_ref.shape[0]
    cm = jnp.max(x_ref[...], axis=2)
    chunk_iota = jax.lax.broadcasted_iota(jnp.int32, cm.shape, 1)
    slot_iota = jax.lax.broadcasted_iota(jnp.int32, (r, _SLOTS), 1)

    def body(j, carry):
        cm_c, idxs = carry
        m = jnp.max(cm_c, axis=1, keepdims=True)
        ii = jnp.min(
            jnp.where(cm_c == m, chunk_iota, _IMAX), axis=1, keepdims=True
        )
        cm_c = jnp.where(chunk_iota == ii, -jnp.inf, cm_c)
        idxs = jnp.where(slot_iota == j, ii, idxs)
        return cm_c, idxs

    _, idxs = jax.lax.fori_loop(
        0, _TOP_K, body, (cm, jnp.zeros((r, _SLOTS), jnp.int32))
    )
    cidx_ref[...] = idxs


def _sampler_kernel(cidx_sref, x_ref, cvec_ref, probs_ref, tok_ref, buf_ref):
    r = x_ref.shape[0]
    vocab = x_ref.shape[1] * x_ref.shape[2]
    i = pl.program_id(0)

    for s in range(_TOP_K):
        for rr in range(_BLOCK_ROWS):
            c = cidx_sref[i * _BLOCK_ROWS + rr, s]
            buf_ref[pl.ds(rr, 1), pl.ds(s * _CHUNK, _CHUNK)] = (
                x_ref[rr, pl.ds(c, 1), :] / _TEMPERATURE
            )

    scaled_buf = buf_ref[...]
    slot_iota = jax.lax.broadcasted_iota(jnp.int32, (r, _SLOTS), 1)

    def body(j, carry):
        vals, cnts = carry
        y = buf_ref[...]
        m = jnp.max(y, axis=1, keepdims=True)
        eq = y == m
        c = jnp.sum(jnp.where(eq, 1.0, 0.0), axis=1, keepdims=True)
        buf_ref[...] = jnp.where(eq, -jnp.inf, y)
        sel = slot_iota == j
        vals = jnp.where(sel, m, vals)
        cnts = jnp.where(sel, c, cnts)
        return vals, cnts

    vals0 = jnp.full((r, _SLOTS), -jnp.inf, jnp.float32)
    cnts0 = jnp.zeros((r, _SLOTS), jnp.float32)
    vals, cnts = jax.lax.fori_loop(0, _TOP_K, body, (vals0, cnts0))

    m_row = vals[:, 0:1]
    wgt = cnts * jnp.exp(vals - m_row)
    cumc = _cumsum_lanes(cnts)
    cumw = _cumsum_lanes(wgt)
    cumc_excl = cumc - cnts
    s_excl = cumw - wgt
    topk_keep = cumc_excl < float(_TOP_K)
    z_topk = jnp.sum(jnp.where(topk_keep, wgt, 0.0), axis=1, keepdims=True)
    keep = jnp.logical_and(topk_keep, s_excl <= _TOP_P * z_topk)
    thresh = jnp.min(jnp.where(keep, vals, jnp.inf), axis=1, keepdims=True)
    z_final = jnp.sum(jnp.where(keep, wgt, 0.0), axis=1, keepdims=True)

    lane_iota = jax.lax.broadcasted_iota(jnp.int32, (r, _BW), 1)
    slot_of = lane_iota // _CHUNK
    lane_in = lane_iota - slot_of * _CHUNK
    cvec = cvec_ref[...]
    chunk_id = jnp.zeros((r, _BW), jnp.int32)
    for s in range(_TOP_K):
        chunk_id = jnp.where(slot_of == s, cvec[:, s : s + 1], chunk_id)
    vidx = chunk_id * _CHUNK + lane_in
    row_glob = i * _BLOCK_ROWS + jax.lax.broadcasted_iota(
        jnp.int32, (r, _BW), 0
    )
    lin = (row_glob * vocab + vidx).astype(jnp.uint32)
    g = _threefry_gumbel(lin)
    log_z = jnp.log(z_final)
    log_eps = jnp.log(jnp.float32(1e-12))
    kept_buf = scaled_buf >= thresh
    score = jnp.where(kept_buf, scaled_buf - m_row - log_z, log_eps) + g
    smax = jnp.max(score, axis=1, keepdims=True)
    tok_ref[...] = jnp.min(
        jnp.where(score == smax, vidx, _IMAX), axis=1, keepdims=True
    )

    scaled = x_ref[...] / _TEMPERATURE
    e = jnp.exp(scaled - m_row[:, :, None])
    kept = scaled >= thresh[:, :, None]
    inv_z = (1.0 / z_final)[:, :, None]
    probs_ref[...] = jnp.where(kept, e * inv_z, 0.0)


@functools.partial(jax.jit)
def kernel(logits):
    rows, vocab = logits.shape
    nchunk = vocab // _CHUNK
    x3 = jnp.reshape(logits, (rows, nchunk, _CHUNK))
    grid = (rows // _BLOCK_ROWS,)

    cidx = pl.pallas_call(
        _chunk_rank_kernel,
        grid=grid,
        in_specs=[pl.BlockSpec((_BLOCK_ROWS, nchunk, _CHUNK), lambda i: (i, 0, 0))],
        out_specs=pl.BlockSpec((_BLOCK_ROWS, _SLOTS), lambda i: (i, 0)),
        out_shape=jax.ShapeDtypeStruct((rows, _SLOTS), jnp.int32),
    )(x3)

    grid_spec = pltpu.PrefetchScalarGridSpec(
        num_scalar_prefetch=1,
        grid=grid,
        in_specs=[
            pl.BlockSpec((_BLOCK_ROWS, nchunk, _CHUNK), lambda i, s: (i, 0, 0)),
            pl.BlockSpec((_BLOCK_ROWS, _SLOTS), lambda i, s: (i, 0)),
        ],
        out_specs=[
            pl.BlockSpec((_BLOCK_ROWS, nchunk, _CHUNK), lambda i, s: (i, 0, 0)),
            pl.BlockSpec((_BLOCK_ROWS, 1), lambda i, s: (i, 0)),
        ],
        scratch_shapes=[pltpu.VMEM((_BLOCK_ROWS, _BW), jnp.float32)],
    )
    probs3, tok = pl.pallas_call(
        _sampler_kernel,
        grid_spec=grid_spec,
        out_shape=[
            jax.ShapeDtypeStruct((rows, nchunk, _CHUNK), jnp.float32),
            jax.ShapeDtypeStruct((rows, 1), jnp.int32),
        ],
    )(cidx, x3, cidx)
    return jnp.reshape(probs3, (rows, vocab)), jnp.reshape(tok, (rows,))
